# Initial kernel scaffold; baseline (speedup 1.0000x reference)
import jax
import jax.numpy as jnp
from jax.experimental import pallas as pl


def kernel(x):
    raise NotImplementedError("write your pallas kernel here")


# speedup vs baseline: 1.2596x; 1.2596x over previous
import functools

import jax
import jax.numpy as jnp
from jax import lax
from jax.experimental import pallas as pl
from jax.experimental.pallas import tpu as pltpu
from jax.experimental.pallas import tpu_sc as plsc

_B, _C, _T, _H, _W = 8, 3, 32, 224, 224
_NF = 4
_STRIDE = 8
_HW = _H * _W
_NBLK = _B * _C * _NF
_NC = 2
_NS = 16
_NW = _NC * _NS
_BLK_PER_W = _NBLK // _NW


def _sc_frame_gather(xf):
    mesh = plsc.VectorSubcoreMesh(core_axis_name="c", subcore_axis_name="s")

    @functools.partial(
        pl.kernel,
        mesh=mesh,
        out_type=jax.ShapeDtypeStruct((_NBLK * _HW,), jnp.float32),
        scratch_types=[pltpu.VMEM((_HW,), jnp.float32)],
    )
    def k(x_hbm, out_hbm, buf):
        wid = lax.axis_index("s") * _NC + lax.axis_index("c")
        for kk in range(_BLK_PER_W):
            g = wid * _BLK_PER_W + kk
            bc = g // _NF
            j = g % _NF
            src = (bc * _T + _STRIDE * j) * _HW
            dst = g * _HW
            pltpu.sync_copy(x_hbm.at[pl.ds(src, _HW)], buf)
            pltpu.sync_copy(buf, out_hbm.at[pl.ds(dst, _HW)])

    return k(xf)


def kernel(x):
    xf = x.reshape(-1)
    out = _sc_frame_gather(xf)
    return out.reshape(_B, _C, _NF, _H, _W)


# speedup vs baseline: 1.2639x; 1.0035x over previous
import functools

import jax
import jax.numpy as jnp
from jax import lax
from jax.experimental import pallas as pl
from jax.experimental.pallas import tpu as pltpu
from jax.experimental.pallas import tpu_sc as plsc

_B, _C, _T, _H, _W = 8, 3, 32, 224, 224
_NF = 4
_STRIDE = 8
_HW = _H * _W
_NBLK = _B * _C * _NF
_NC = 2
_NS = 16
_NW = _NC * _NS
_BLK_PER_W = _NBLK // _NW


def _sc_frame_gather(xf):
    mesh = plsc.VectorSubcoreMesh(core_axis_name="c", subcore_axis_name="s")

    @functools.partial(
        pl.kernel,
        mesh=mesh,
        out_type=jax.ShapeDtypeStruct((_NBLK * _HW,), jnp.float32),
        scratch_types=[
            pltpu.VMEM((_HW,), jnp.float32),
            pltpu.VMEM((_HW,), jnp.float32),
            pltpu.SemaphoreType.DMA,
            pltpu.SemaphoreType.DMA,
            pltpu.SemaphoreType.DMA,
            pltpu.SemaphoreType.DMA,
        ],
    )
    def k(x_hbm, out_hbm, buf0, buf1, si0, si1, so0, so1):
        wid = lax.axis_index("s") * _NC + lax.axis_index("c")
        bufs = (buf0, buf1)
        sis = (si0, si1)
        sos = (so0, so1)

        def offs(kk):
            g = wid * _BLK_PER_W + kk
            bc = g // _NF
            j = g % _NF
            return (bc * _T + _STRIDE * j) * _HW, g * _HW

        gathers = [None, None]
        scatters = [None, None]
        for kk in range(_BLK_PER_W):
            s = kk % 2
            src, dst = offs(kk)
            if scatters[s] is not None:
                scatters[s].wait()
            gathers[s] = pltpu.async_copy(
                x_hbm.at[pl.ds(src, _HW)], bufs[s], sis[s]
            )
            if kk >= 1:
                p = (kk - 1) % 2
                gathers[p].wait()
                _, pdst = offs(kk - 1)
                scatters[p] = pltpu.async_copy(
                    bufs[p], out_hbm.at[pl.ds(pdst, _HW)], sos[p]
                )
        last = (_BLK_PER_W - 1) % 2
        gathers[last].wait()
        _, ldst = offs(_BLK_PER_W - 1)
        scatters[last] = pltpu.async_copy(
            bufs[last], out_hbm.at[pl.ds(ldst, _HW)], sos[last]
        )
        for s in range(2):
            if scatters[s] is not None:
                scatters[s].wait()

    return k(xf)


def kernel(x):
    xf = x.reshape(-1)
    out = _sc_frame_gather(xf)
    return out.reshape(_B, _C, _NF, _H, _W)
